# Initial kernel scaffold; baseline (speedup 1.0000x reference)
import jax
import jax.numpy as jnp
from jax.experimental import pallas as pl


def kernel(x, table):
    raise NotImplementedError("write your pallas kernel here")


# speedup vs baseline: 1.1116x; 1.1116x over previous
import functools

import jax
import jax.numpy as jnp
from jax import lax
from jax.experimental import pallas as pl
from jax.experimental.pallas import tpu as pltpu
from jax.experimental.pallas import tpu_sc as plsc

NUM_CORES = 2
NUM_SUBCORES = 16
NUM_WORKERS = NUM_CORES * NUM_SUBCORES

CHUNK = 3200


@functools.partial(jax.jit, static_argnames=("n_idx", "dim"))
def _sc_gather(x_flat, table, n_idx, dim):
    per_w = n_idx // NUM_WORKERS
    n_chunks = per_w // CHUNK
    mesh = plsc.VectorSubcoreMesh(core_axis_name="c", subcore_axis_name="s")

    @functools.partial(
        pl.kernel,
        mesh=mesh,
        out_type=jax.ShapeDtypeStruct((n_idx, dim), jnp.float32),
        scratch_types=[
            pltpu.VMEM((CHUNK,), jnp.int32),
            pltpu.VMEM((CHUNK, dim), jnp.float32),
            pltpu.SemaphoreType.DMA,
        ],
        compiler_params=pltpu.CompilerParams(use_tc_tiling_on_sc=False),
    )
    def k(x_hbm, table_hbm, out_hbm, idx_v, rows_v, sem):
        wid = lax.axis_index("s") * NUM_CORES + lax.axis_index("c")
        base = pl.multiple_of(wid * per_w, CHUNK)

        def body(i, _):
            off = pl.multiple_of(base + i * CHUNK, CHUNK)
            pltpu.sync_copy(x_hbm.at[pl.ds(off, CHUNK)], idx_v)
            pltpu.async_copy(table_hbm.at[idx_v], rows_v, sem).wait()
            pltpu.sync_copy(rows_v, out_hbm.at[pl.ds(off, CHUNK)])
            return 0

        lax.fori_loop(0, n_chunks, body, 0)

    return k(x_flat, table)


def kernel(x, table):
    b, l = x.shape
    dim = table.shape[1]
    x_flat = x.reshape(b * l).astype(jnp.int32)
    out = _sc_gather(x_flat, table, b * l, dim)
    return out.reshape(b, l, dim)


# speedup vs baseline: 1.1142x; 1.0023x over previous
import functools

import jax
import jax.numpy as jnp
from jax import lax
from jax.experimental import pallas as pl
from jax.experimental.pallas import tpu as pltpu
from jax.experimental.pallas import tpu_sc as plsc

NUM_CORES = 2
NUM_SUBCORES = 16
NUM_WORKERS = NUM_CORES * NUM_SUBCORES

CHUNK = 1600
NBUF = 2


@functools.partial(jax.jit, static_argnames=("n_idx", "dim"))
def _sc_gather(x_flat, table, n_idx, dim):
    per_w = n_idx // NUM_WORKERS
    n_chunks = per_w // CHUNK
    n_outer = n_chunks // NBUF
    mesh = plsc.VectorSubcoreMesh(core_axis_name="c", subcore_axis_name="s")

    @functools.partial(
        pl.kernel,
        mesh=mesh,
        out_type=jax.ShapeDtypeStruct((n_idx, dim), jnp.float32),
        scratch_types=[
            pltpu.VMEM((per_w,), jnp.int32),
            [pltpu.VMEM((CHUNK, dim), jnp.float32) for _ in range(NBUF)],
            [pltpu.SemaphoreType.DMA for _ in range(NBUF)],
            [pltpu.SemaphoreType.DMA for _ in range(NBUF)],
        ],
        compiler_params=pltpu.CompilerParams(use_tc_tiling_on_sc=False),
    )
    def k(x_hbm, table_hbm, out_hbm, idx_v, rows, gsem, wsem):
        wid = lax.axis_index("s") * NUM_CORES + lax.axis_index("c")
        base = pl.multiple_of(wid * per_w, per_w)
        pltpu.sync_copy(x_hbm.at[pl.ds(base, per_w)], idx_v)

        def gather(chunk_i, b):
            off = pl.multiple_of(chunk_i * CHUNK, CHUNK)
            pltpu.async_copy(
                table_hbm.at[idx_v.at[pl.ds(off, CHUNK)]], rows[b], gsem[b]
            )

        def writeback(chunk_i, b):
            off = pl.multiple_of(base + chunk_i * CHUNK, CHUNK)
            pltpu.async_copy(rows[b], out_hbm.at[pl.ds(off, CHUNK)], wsem[b])

        for b in range(NBUF):
            gather(b, b)

        def body(g, _):
            i0 = g * NBUF
            for b in range(NBUF):
                i = i0 + b
                pltpu.make_async_copy(
                    table_hbm.at[idx_v.at[pl.ds(0, CHUNK)]], rows[b], gsem[b]
                ).wait()
                writeback(i, b)
                @pl.when(i + NBUF < n_chunks)
                def _():
                    pltpu.make_async_copy(
                        rows[b], out_hbm.at[pl.ds(base, CHUNK)], wsem[b]
                    ).wait()
                    gather(i + NBUF, b)
            return 0

        lax.fori_loop(0, n_outer, body, 0)

        for b in range(NBUF):
            pltpu.make_async_copy(
                rows[b], out_hbm.at[pl.ds(base, CHUNK)], wsem[b]
            ).wait()

    return k(x_flat, table)


def kernel(x, table):
    b, l = x.shape
    dim = table.shape[1]
    x_flat = x.reshape(b * l).astype(jnp.int32)
    out = _sc_gather(x_flat, table, b * l, dim)
    return out.reshape(b, l, dim)
